# baseline (speedup 1.0000x reference)
import functools

import jax
import jax.numpy as jnp
from jax import lax
from jax.experimental import pallas as pl
from jax.experimental.pallas import tpu as pltpu
from jax.experimental.pallas import tpu_sc as plsc


_INFO = plsc.get_sparse_core_info()
_NW = _INFO.num_cores * _INFO.num_subcores
_L = _INFO.num_lanes


@functools.partial(jax.jit, static_argnames=("batch", "hist", "dim"))
def _gather_rows(flat_idx, table16, batch, hist, dim):
    bw = batch // _NW
    mesh = plsc.VectorSubcoreMesh(core_axis_name="c", subcore_axis_name="s")

    @functools.partial(
        pl.kernel,
        mesh=mesh,
        out_type=jax.ShapeDtypeStruct((hist, dim, batch), jnp.float32),
        scratch_types=[
            pltpu.VMEM((bw * hist,), jnp.int32),
            pltpu.VMEM((2 * bw,), jnp.int32),
            pltpu.VMEM((2 * bw,), jnp.int32),
            pltpu.VMEM((2 * bw, _L), jnp.float32),
            pltpu.VMEM((2 * bw, _L), jnp.float32),
            pltpu.VMEM((1, dim, bw), jnp.float32),
            pltpu.VMEM((1, dim, bw), jnp.float32),
            pltpu.SemaphoreType.DMA,
            pltpu.SemaphoreType.DMA,
            pltpu.SemaphoreType.DMA,
            pltpu.SemaphoreType.DMA,
            pltpu.SemaphoreType.DMA,
        ],
        compiler_params=pltpu.CompilerParams(
            use_tc_tiling_on_sc=False, needs_layout_passes=False
        ),
    )
    def k(idx_hbm, table_hbm, out_hbm, iv_all, ev0, ev1, rv0, rv1, tv0, tv1,
          sa, sg0, sg1, so0, so1):
        eidx_v = (ev0, ev1)
        rows_v = (rv0, rv1)
        stage_v = (tv0, tv1)
        sg = (sg0, sg1)
        so = (so0, so1)
        wid = lax.axis_index("s") * _INFO.num_cores + lax.axis_index("c")
        b0 = pl.multiple_of(wid * bw, 8)
        lanes = lax.iota(jnp.int32, _L)

        def expand(h, s):
            def body(j, carry):
                src = (lanes + j * _L) * hist + h
                v = plsc.load_gather(iv_all, [src])
                v2 = v * 2
                pos = lanes * 2 + 2 * j * _L
                plsc.store_scatter(eidx_v[s], [pos], v2)
                plsc.store_scatter(eidx_v[s], [pos + 1], v2 + 1)
                return carry

            lax.fori_loop(0, bw // _L, body, 0)

        def gat_start(s):
            return pltpu.async_copy(table_hbm.at[eidx_v[s]], rows_v[s], sg[s])

        def transpose(s):
            def body(jb, carry):
                base = lanes * 2 + jb * 2 * _L
                for d in range(dim):
                    val = plsc.load_gather(
                        rows_v[s], [base + (d // _L), jnp.full((_L,), d % _L, jnp.int32)]
                    )
                    stage_v[s][0, d, pl.ds(pl.multiple_of(jb * _L, 8), _L)] = val
                return carry

            lax.fori_loop(0, bw // _L, body, 0)

        def out_start(h, s):
            return pltpu.async_copy(
                stage_v[s], out_hbm.at[pl.ds(h, 1), :, pl.ds(b0, bw)], so[s]
            )

        def out_wait(h, s):
            pltpu.make_async_copy(
                stage_v[s], out_hbm.at[pl.ds(h, 1), :, pl.ds(b0, bw)], so[s]
            ).wait()

        def gat_wait(s):
            pltpu.make_async_copy(table_hbm.at[eidx_v[s]], rows_v[s], sg[s]).wait()

        pltpu.async_copy(
            idx_hbm.at[pl.ds(pl.multiple_of(b0 * hist, 8), bw * hist)], iv_all, sa
        ).wait()

        expand(0, 0)
        gat_start(0)
        gat_wait(0)
        expand(1, 1)
        gat_start(1)
        transpose(0)
        out_start(0, 0)
        gat_wait(1)
        expand(2, 0)
        gat_start(0)
        transpose(1)
        out_start(1, 1)

        def steady(kk, carry):
            h = 2 * kk
            for q in (0, 1):
                hh = h + q
                s = q
                gat_wait(s)
                expand(hh + 1, s ^ 1)
                gat_start(s ^ 1)
                out_wait(hh - 2, s)
                transpose(s)
                out_start(hh, s)
            return carry

        lax.fori_loop(1, (hist - 2) // 2, steady, 0)
        s = 0
        gat_wait(s)
        expand(hist - 1, s ^ 1)
        gat_start(s ^ 1)
        out_wait(hist - 4, s)
        transpose(s)
        out_start(hist - 2, s)
        s = 1
        gat_wait(s)
        out_wait(hist - 3, s)
        transpose(s)
        out_start(hist - 1, s)
        out_wait(hist - 2, 0)
        out_wait(hist - 1, 1)

    return k(flat_idx, table16)


def kernel(indices, table):
    b, h = indices.shape
    v, d = table.shape
    flat = indices.reshape(b * h).astype(jnp.int32)
    table16 = table.reshape(v * d // _L, _L)
    out = _gather_rows(flat, table16, b, h, d)
    return jnp.transpose(out, (2, 0, 1))


# speedup vs baseline: 1.0000x; 1.0000x over previous
import functools

import jax
import jax.numpy as jnp
from jax import lax
from jax.experimental import pallas as pl
from jax.experimental.pallas import tpu as pltpu
from jax.experimental.pallas import tpu_sc as plsc


_INFO = plsc.get_sparse_core_info()
_NW = _INFO.num_cores * _INFO.num_subcores
_L = _INFO.num_lanes


@functools.partial(jax.jit, static_argnames=("batch", "hist", "dim"))
def _gather_rows(flat_idx, table16, batch, hist, dim):
    bw = batch // _NW
    mesh = plsc.VectorSubcoreMesh(core_axis_name="c", subcore_axis_name="s")

    @functools.partial(
        pl.kernel,
        mesh=mesh,
        out_type=jax.ShapeDtypeStruct((hist, dim, batch), jnp.float32),
        scratch_types=[
            pltpu.VMEM((bw * hist,), jnp.int32),
            pltpu.VMEM((2 * bw,), jnp.int32),
            pltpu.VMEM((2 * bw,), jnp.int32),
            pltpu.VMEM((2 * bw, _L), jnp.float32),
            pltpu.VMEM((2 * bw, _L), jnp.float32),
            pltpu.VMEM((1, dim, bw), jnp.float32),
            pltpu.VMEM((1, dim, bw), jnp.float32),
            pltpu.SemaphoreType.DMA,
            pltpu.SemaphoreType.DMA,
            pltpu.SemaphoreType.DMA,
            pltpu.SemaphoreType.DMA,
            pltpu.SemaphoreType.DMA,
        ],
        compiler_params=pltpu.CompilerParams(
            use_tc_tiling_on_sc=False, needs_layout_passes=False
        ),
    )
    def k(idx_hbm, table_hbm, out_hbm, iv_all, ev0, ev1, rv0, rv1, tv0, tv1,
          sa, sg0, sg1, so0, so1):
        eidx_v = (ev0, ev1)
        rows_v = (rv0, rv1)
        stage_v = (tv0, tv1)
        sg = (sg0, sg1)
        so = (so0, so1)
        wid = lax.axis_index("s") * _INFO.num_cores + lax.axis_index("c")
        b0 = pl.multiple_of(wid * bw, 8)
        lanes = lax.iota(jnp.int32, _L)

        def expand(h, s):
            def body(j, carry):
                src = (lanes + j * _L) * hist + h
                v = plsc.load_gather(iv_all, [src])
                v2 = v * 2
                pos = lanes * 2 + 2 * j * _L
                plsc.store_scatter(eidx_v[s], [pos], v2)
                plsc.store_scatter(eidx_v[s], [pos + 1], v2 + 1)
                return carry

            lax.fori_loop(0, bw // _L, body, 0)

        def gat_start(s):
            return pltpu.async_copy(table_hbm.at[eidx_v[s]], rows_v[s], sg[s])

        lanes2 = lanes * 2
        cols = [jnp.full((_L,), c, jnp.int32) for c in range(_L)]

        def transpose(s):
            def body(jb, carry):
                base = lanes2 + jb * 2 * _L
                rvecs = (base, base + 1)
                dst = pl.ds(pl.multiple_of(jb * _L, 8), _L)
                for d in range(dim):
                    val = plsc.load_gather(rows_v[s], [rvecs[d // _L], cols[d % _L]])
                    stage_v[s][0, d, dst] = val
                return carry

            lax.fori_loop(0, bw // _L, body, 0)

        def out_start(h, s):
            return pltpu.async_copy(
                stage_v[s], out_hbm.at[pl.ds(h, 1), :, pl.ds(b0, bw)], so[s]
            )

        def out_wait(h, s):
            pltpu.make_async_copy(
                stage_v[s], out_hbm.at[pl.ds(h, 1), :, pl.ds(b0, bw)], so[s]
            ).wait()

        def gat_wait(s):
            pltpu.make_async_copy(table_hbm.at[eidx_v[s]], rows_v[s], sg[s]).wait()

        pltpu.async_copy(
            idx_hbm.at[pl.ds(pl.multiple_of(b0 * hist, 8), bw * hist)], iv_all, sa
        ).wait()

        expand(0, 0)
        gat_start(0)
        gat_wait(0)
        expand(1, 1)
        gat_start(1)
        transpose(0)
        out_start(0, 0)
        gat_wait(1)
        expand(2, 0)
        gat_start(0)
        transpose(1)
        out_start(1, 1)

        def steady(kk, carry):
            h = 2 * kk
            for q in (0, 1):
                hh = h + q
                s = q
                gat_wait(s)
                expand(hh + 1, s ^ 1)
                gat_start(s ^ 1)
                out_wait(hh - 2, s)
                transpose(s)
                out_start(hh, s)
            return carry

        lax.fori_loop(1, (hist - 2) // 2, steady, 0)
        s = 0
        gat_wait(s)
        expand(hist - 1, s ^ 1)
        gat_start(s ^ 1)
        out_wait(hist - 4, s)
        transpose(s)
        out_start(hist - 2, s)
        s = 1
        gat_wait(s)
        out_wait(hist - 3, s)
        transpose(s)
        out_start(hist - 1, s)
        out_wait(hist - 2, 0)
        out_wait(hist - 1, 1)

    return k(flat_idx, table16)


def kernel(indices, table):
    b, h = indices.shape
    v, d = table.shape
    flat = indices.reshape(b * h).astype(jnp.int32)
    table16 = table.reshape(v * d // _L, _L)
    out = _gather_rows(flat, table16, b, h, d)
    return jnp.transpose(out, (2, 0, 1))


# speedup vs baseline: 1.4009x; 1.4008x over previous
import functools

import jax
import jax.numpy as jnp
from jax import lax
from jax.experimental import pallas as pl
from jax.experimental.pallas import tpu as pltpu
from jax.experimental.pallas import tpu_sc as plsc


_INFO = plsc.get_sparse_core_info()
_NW = _INFO.num_cores * _INFO.num_subcores
_L = _INFO.num_lanes


@functools.partial(jax.jit, static_argnames=("batch", "hist", "dim"))
def _gather_rows(flat_idx, table16, batch, hist, dim):
    bw = batch // _NW
    mesh = plsc.VectorSubcoreMesh(core_axis_name="c", subcore_axis_name="s")

    @functools.partial(
        pl.kernel,
        mesh=mesh,
        out_type=jax.ShapeDtypeStruct((hist, dim, batch), jnp.float32),
        scratch_types=[
            pltpu.VMEM((bw * hist,), jnp.int32),
            pltpu.VMEM((2 * bw,), jnp.int32),
            pltpu.VMEM((2 * bw,), jnp.int32),
            pltpu.VMEM((2 * bw, _L), jnp.float32),
            pltpu.VMEM((2 * bw, _L), jnp.float32),
            pltpu.VMEM((1, dim, bw + 9), jnp.float32),
            pltpu.VMEM((1, dim, bw + 9), jnp.float32),
            pltpu.SemaphoreType.DMA,
            pltpu.SemaphoreType.DMA,
            pltpu.SemaphoreType.DMA,
            pltpu.SemaphoreType.DMA,
            pltpu.SemaphoreType.DMA,
        ],
        compiler_params=pltpu.CompilerParams(
            use_tc_tiling_on_sc=False, needs_layout_passes=False
        ),
    )
    def k(idx_hbm, table_hbm, out_hbm, iv_all, ev0, ev1, rv0, rv1, tv0, tv1,
          sa, sg0, sg1, so0, so1):
        eidx_v = (ev0, ev1)
        rows_v = (rv0, rv1)
        stage_v = (tv0, tv1)
        sg = (sg0, sg1)
        so = (so0, so1)
        wid = lax.axis_index("s") * _INFO.num_cores + lax.axis_index("c")
        b0 = pl.multiple_of(wid * bw, 8)
        lanes = lax.iota(jnp.int32, _L)

        def expand(h, s):
            def body(j, carry):
                src = (lanes + j * _L) * hist + h
                v = plsc.load_gather(iv_all, [src])
                v2 = v * 2
                pos = lanes * 2 + 2 * j * _L
                plsc.store_scatter(eidx_v[s], [pos], v2)
                plsc.store_scatter(eidx_v[s], [pos + 1], v2 + 1)
                return carry

            lax.fori_loop(0, bw // _L, body, 0)

        def gat_start(s):
            return pltpu.async_copy(table_hbm.at[eidx_v[s]], rows_v[s], sg[s])

        zv = lanes * 0
        lanes16 = lanes + _L

        def transpose(s):
            def body(jb, carry):
                for jj in range(_L):
                    j = jb * _L + jj
                    jv = zv + j
                    row0 = plsc.load_gather(rows_v[s], [jv + jv, lanes])
                    row1 = plsc.load_gather(rows_v[s], [jv + jv + 1, lanes])
                    plsc.store_scatter(stage_v[s], [zv, lanes, jv], row0)
                    plsc.store_scatter(stage_v[s], [zv, lanes16, jv], row1)
                return carry

            lax.fori_loop(0, bw // _L, body, 0)

        def out_start(h, s):
            return pltpu.async_copy(
                stage_v[s].at[:, :, pl.ds(0, bw)],
                out_hbm.at[pl.ds(h, 1), :, pl.ds(b0, bw)],
                so[s],
            )

        def out_wait(h, s):
            pltpu.make_async_copy(
                stage_v[s].at[:, :, pl.ds(0, bw)],
                out_hbm.at[pl.ds(h, 1), :, pl.ds(b0, bw)],
                so[s],
            ).wait()

        def gat_wait(s):
            pltpu.make_async_copy(table_hbm.at[eidx_v[s]], rows_v[s], sg[s]).wait()

        pltpu.async_copy(
            idx_hbm.at[pl.ds(pl.multiple_of(b0 * hist, 8), bw * hist)], iv_all, sa
        ).wait()

        expand(0, 0)
        gat_start(0)
        gat_wait(0)
        expand(1, 1)
        gat_start(1)
        transpose(0)
        out_start(0, 0)
        gat_wait(1)
        expand(2, 0)
        gat_start(0)
        transpose(1)
        out_start(1, 1)

        def steady(kk, carry):
            h = 2 * kk
            for q in (0, 1):
                hh = h + q
                s = q
                gat_wait(s)
                expand(hh + 1, s ^ 1)
                gat_start(s ^ 1)
                out_wait(hh - 2, s)
                transpose(s)
                out_start(hh, s)
            return carry

        lax.fori_loop(1, (hist - 2) // 2, steady, 0)
        s = 0
        gat_wait(s)
        expand(hist - 1, s ^ 1)
        gat_start(s ^ 1)
        out_wait(hist - 4, s)
        transpose(s)
        out_start(hist - 2, s)
        s = 1
        gat_wait(s)
        out_wait(hist - 3, s)
        transpose(s)
        out_start(hist - 1, s)
        out_wait(hist - 2, 0)
        out_wait(hist - 1, 1)

    return k(flat_idx, table16)


def kernel(indices, table):
    b, h = indices.shape
    v, d = table.shape
    flat = indices.reshape(b * h).astype(jnp.int32)
    table16 = table.reshape(v * d // _L, _L)
    out = _gather_rows(flat, table16, b, h, d)
    return jnp.transpose(out, (2, 0, 1))


# speedup vs baseline: 1.4040x; 1.0022x over previous
import functools

import jax
import jax.numpy as jnp
from jax import lax
from jax.experimental import pallas as pl
from jax.experimental.pallas import tpu as pltpu
from jax.experimental.pallas import tpu_sc as plsc


_INFO = plsc.get_sparse_core_info()
_NW = _INFO.num_cores * _INFO.num_subcores
_L = _INFO.num_lanes


@functools.partial(jax.jit, static_argnames=("batch", "hist", "dim"))
def _gather_rows(flat_idx, table16, batch, hist, dim):
    bw = batch // _NW
    mesh = plsc.VectorSubcoreMesh(core_axis_name="c", subcore_axis_name="s")

    @functools.partial(
        pl.kernel,
        mesh=mesh,
        out_type=jax.ShapeDtypeStruct((hist, dim, batch), jnp.float32),
        scratch_types=[
            pltpu.VMEM((hist, bw), jnp.int32),
            pltpu.VMEM((2 * bw,), jnp.int32),
            pltpu.VMEM((2 * bw,), jnp.int32),
            pltpu.VMEM((2 * bw, _L), jnp.float32),
            pltpu.VMEM((2 * bw, _L), jnp.float32),
            pltpu.VMEM((1, dim, bw + 9), jnp.float32),
            pltpu.VMEM((1, dim, bw + 9), jnp.float32),
            pltpu.SemaphoreType.DMA,
            pltpu.SemaphoreType.DMA,
            pltpu.SemaphoreType.DMA,
            pltpu.SemaphoreType.DMA,
            pltpu.SemaphoreType.DMA,
        ],
        compiler_params=pltpu.CompilerParams(
            use_tc_tiling_on_sc=False, needs_layout_passes=False
        ),
    )
    def k(idx_hbm, table_hbm, out_hbm, iv_all, ev0, ev1, rv0, rv1, tv0, tv1,
          sa, sg0, sg1, so0, so1):
        eidx_v = (ev0, ev1)
        rows_v = (rv0, rv1)
        stage_v = (tv0, tv1)
        sg = (sg0, sg1)
        so = (so0, so1)
        wid = lax.axis_index("s") * _INFO.num_cores + lax.axis_index("c")
        b0 = pl.multiple_of(wid * bw, 8)
        lanes = lax.iota(jnp.int32, _L)

        def expand(h, s):
            hv = lanes * 0 + h

            def body(j, carry):
                v = plsc.load_gather(iv_all, [hv, lanes + j * _L])
                v2 = v * 2
                pos = lanes * 2 + 2 * j * _L
                plsc.store_scatter(eidx_v[s], [pos], v2)
                plsc.store_scatter(eidx_v[s], [pos + 1], v2 + 1)
                return carry

            lax.fori_loop(0, bw // _L, body, 0)

        def gat_start(s):
            return pltpu.async_copy(table_hbm.at[eidx_v[s]], rows_v[s], sg[s])

        zv = lanes * 0
        lanes16 = lanes + _L

        def transpose(s):
            def body(jb, carry):
                for jj in range(_L):
                    j = jb * _L + jj
                    jv = zv + j
                    row0 = plsc.load_gather(rows_v[s], [jv + jv, lanes])
                    row1 = plsc.load_gather(rows_v[s], [jv + jv + 1, lanes])
                    plsc.store_scatter(stage_v[s], [zv, lanes, jv], row0)
                    plsc.store_scatter(stage_v[s], [zv, lanes16, jv], row1)
                return carry

            lax.fori_loop(0, bw // _L, body, 0)

        def out_start(h, s):
            return pltpu.async_copy(
                stage_v[s].at[:, :, pl.ds(0, bw)],
                out_hbm.at[pl.ds(h, 1), :, pl.ds(b0, bw)],
                so[s],
            )

        def out_wait(h, s):
            pltpu.make_async_copy(
                stage_v[s].at[:, :, pl.ds(0, bw)],
                out_hbm.at[pl.ds(h, 1), :, pl.ds(b0, bw)],
                so[s],
            ).wait()

        def gat_wait(s):
            pltpu.make_async_copy(table_hbm.at[eidx_v[s]], rows_v[s], sg[s]).wait()

        pltpu.async_copy(
            idx_hbm.at[:, pl.ds(b0, bw)], iv_all, sa
        ).wait()

        expand(0, 0)
        gat_start(0)
        gat_wait(0)
        expand(1, 1)
        gat_start(1)
        transpose(0)
        out_start(0, 0)
        gat_wait(1)
        expand(2, 0)
        gat_start(0)
        transpose(1)
        out_start(1, 1)

        def steady(kk, carry):
            h = 2 * kk
            for q in (0, 1):
                hh = h + q
                s = q
                gat_wait(s)
                expand(hh + 1, s ^ 1)
                gat_start(s ^ 1)
                out_wait(hh - 2, s)
                transpose(s)
                out_start(hh, s)
            return carry

        lax.fori_loop(1, (hist - 2) // 2, steady, 0)
        s = 0
        gat_wait(s)
        expand(hist - 1, s ^ 1)
        gat_start(s ^ 1)
        out_wait(hist - 4, s)
        transpose(s)
        out_start(hist - 2, s)
        s = 1
        gat_wait(s)
        out_wait(hist - 3, s)
        transpose(s)
        out_start(hist - 1, s)
        out_wait(hist - 2, 0)
        out_wait(hist - 1, 1)

    return k(flat_idx, table16)


def kernel(indices, table):
    b, h = indices.shape
    v, d = table.shape
    idxt = jnp.transpose(indices).astype(jnp.int32)
    table16 = table.reshape(v * d // _L, _L)
    out = _gather_rows(idxt, table16, b, h, d)
    return jnp.transpose(out, (2, 0, 1))


# speedup vs baseline: 1.4103x; 1.0045x over previous
import functools

import jax
import jax.numpy as jnp
from jax import lax
from jax.experimental import pallas as pl
from jax.experimental.pallas import tpu as pltpu
from jax.experimental.pallas import tpu_sc as plsc


_INFO = plsc.get_sparse_core_info()
_NW = _INFO.num_cores * _INFO.num_subcores
_L = _INFO.num_lanes


@functools.partial(jax.jit, static_argnames=("batch", "hist", "dim"))
def _gather_rows(flat_idx, table16, batch, hist, dim):
    bw = batch // _NW
    mesh = plsc.VectorSubcoreMesh(core_axis_name="c", subcore_axis_name="s")

    @functools.partial(
        pl.kernel,
        mesh=mesh,
        out_type=jax.ShapeDtypeStruct((hist, dim, batch), jnp.float32),
        scratch_types=[
            pltpu.VMEM((hist, bw), jnp.int32),
            pltpu.VMEM((bw,), jnp.int32),
            pltpu.VMEM((bw,), jnp.int32),
            pltpu.VMEM((bw, 2 * _L), jnp.float32),
            pltpu.VMEM((bw, 2 * _L), jnp.float32),
            pltpu.VMEM((1, dim, bw + 9), jnp.float32),
            pltpu.VMEM((1, dim, bw + 9), jnp.float32),
            pltpu.SemaphoreType.DMA,
            pltpu.SemaphoreType.DMA,
            pltpu.SemaphoreType.DMA,
            pltpu.SemaphoreType.DMA,
            pltpu.SemaphoreType.DMA,
        ],
        compiler_params=pltpu.CompilerParams(
            use_tc_tiling_on_sc=False, needs_layout_passes=False
        ),
    )
    def k(idx_hbm, table_hbm, out_hbm, iv_all, ev0, ev1, rv0, rv1, tv0, tv1,
          sa, sg0, sg1, so0, so1):
        eidx_v = (ev0, ev1)
        rows_v = (rv0, rv1)
        stage_v = (tv0, tv1)
        sg = (sg0, sg1)
        so = (so0, so1)
        wid = lax.axis_index("s") * _INFO.num_cores + lax.axis_index("c")
        b0 = pl.multiple_of(wid * bw, 8)
        lanes = lax.iota(jnp.int32, _L)

        def expand(h, s):
            hv = lanes * 0 + h

            def body(j, carry):
                v = plsc.load_gather(iv_all, [hv, lanes + j * _L])
                eidx_v[s][pl.ds(pl.multiple_of(j * _L, 8), _L)] = v
                return carry

            lax.fori_loop(0, bw // _L, body, 0)

        def gat_start(s):
            return pltpu.async_copy(table_hbm.at[eidx_v[s]], rows_v[s], sg[s])

        zv = lanes * 0
        lanes16 = lanes + _L

        def transpose(s):
            def body(jb, carry):
                for jj in range(_L):
                    j = jb * _L + jj
                    jv = zv + j
                    row0 = plsc.load_gather(rows_v[s], [jv, lanes])
                    row1 = plsc.load_gather(rows_v[s], [jv, lanes16])
                    plsc.store_scatter(stage_v[s], [zv, lanes, jv], row0)
                    plsc.store_scatter(stage_v[s], [zv, lanes16, jv], row1)
                return carry

            lax.fori_loop(0, bw // _L, body, 0)

        def out_start(h, s):
            return pltpu.async_copy(
                stage_v[s].at[:, :, pl.ds(0, bw)],
                out_hbm.at[pl.ds(h, 1), :, pl.ds(b0, bw)],
                so[s],
            )

        def out_wait(h, s):
            pltpu.make_async_copy(
                stage_v[s].at[:, :, pl.ds(0, bw)],
                out_hbm.at[pl.ds(h, 1), :, pl.ds(b0, bw)],
                so[s],
            ).wait()

        def gat_wait(s):
            pltpu.make_async_copy(table_hbm.at[eidx_v[s]], rows_v[s], sg[s]).wait()

        pltpu.async_copy(
            idx_hbm.at[:, pl.ds(b0, bw)], iv_all, sa
        ).wait()

        expand(0, 0)
        gat_start(0)
        gat_wait(0)
        expand(1, 1)
        gat_start(1)
        transpose(0)
        out_start(0, 0)
        gat_wait(1)
        expand(2, 0)
        gat_start(0)
        transpose(1)
        out_start(1, 1)

        def steady(kk, carry):
            h = 2 * kk
            for q in (0, 1):
                hh = h + q
                s = q
                gat_wait(s)
                expand(hh + 1, s ^ 1)
                gat_start(s ^ 1)
                out_wait(hh - 2, s)
                transpose(s)
                out_start(hh, s)
            return carry

        lax.fori_loop(1, (hist - 2) // 2, steady, 0)
        s = 0
        gat_wait(s)
        expand(hist - 1, s ^ 1)
        gat_start(s ^ 1)
        out_wait(hist - 4, s)
        transpose(s)
        out_start(hist - 2, s)
        s = 1
        gat_wait(s)
        out_wait(hist - 3, s)
        transpose(s)
        out_start(hist - 1, s)
        out_wait(hist - 2, 0)
        out_wait(hist - 1, 1)

    return k(flat_idx, table16)


def kernel(indices, table):
    b, h = indices.shape
    v, d = table.shape
    idxt = jnp.transpose(indices).astype(jnp.int32)
    out = _gather_rows(idxt, table, b, h, d)
    return jnp.transpose(out, (2, 0, 1))
